# baseline (speedup 1.0000x reference)
import functools

import jax
import jax.numpy as jnp
from jax.experimental import pallas as pl
from jax.experimental.pallas import tpu as pltpu

_N = 4096
_M = 4096
_TILE = 256
_STEPS = 10
_SWEEPS3 = 6


def _bf(x):
    return x.astype(jnp.bfloat16).astype(jnp.float32)


def _jacobi3(A):
    a = dict(A)
    V = [[jnp.float32(1.0) if i == j else jnp.float32(0.0) for j in range(3)]
         for i in range(3)]

    def get(i, j):
        return a[(i, j)] if i <= j else a[(j, i)]

    def put(i, j, v):
        a[(i, j) if i <= j else (j, i)] = v

    for _ in range(_SWEEPS3):
        for (p, q) in ((0, 1), (0, 2), (1, 2)):
            apq = get(p, q)
            app = get(p, p)
            aqq = get(q, q)
            small = jnp.abs(apq) < 1e-30
            apq_s = jnp.where(small, 1.0, apq)
            theta = (aqq - app) / (2.0 * apq_s)
            r = jnp.sqrt(theta * theta + 1.0)
            t = jnp.where(theta >= 0, 1.0 / (theta + r), -1.0 / (r - theta))
            t = jnp.where(small, 0.0, t)
            c = jax.lax.rsqrt(t * t + 1.0)
            s = t * c
            k = 3 - p - q
            akp = get(k, p)
            akq = get(k, q)
            put(k, p, c * akp - s * akq)
            put(k, q, s * akp + c * akq)
            put(p, p, app - t * apq)
            put(q, q, aqq + t * apq)
            put(p, q, jnp.float32(0.0))
            for kk in range(3):
                vkp = V[kk][p]
                vkq = V[kk][q]
                V[kk][p] = c * vkp - s * vkq
                V[kk][q] = s * vkp + c * vkq
    return [get(i, i) for i in range(3)], V


def _solve_rt(H, mu_s, mu_t):
    A = {}
    for i in range(3):
        for j in range(i, 3):
            A[(i, j)] = (H[0][i] * H[0][j] + H[1][i] * H[1][j]) \
                + H[2][i] * H[2][j]
    lam, V = _jacobi3(A)

    def cs(i, j, lam, V):
        sw = lam[j] > lam[i]
        li = jnp.where(sw, lam[j], lam[i])
        lj = jnp.where(sw, lam[i], lam[j])
        lam = list(lam)
        lam[i], lam[j] = li, lj
        V = [row[:] for row in V]
        for r in range(3):
            vi = jnp.where(sw, V[r][j], V[r][i])
            vj = jnp.where(sw, V[r][i], V[r][j])
            V[r][i], V[r][j] = vi, vj
        return lam, V

    lam, V = cs(0, 1, lam, V)
    lam, V = cs(0, 2, lam, V)
    lam, V = cs(1, 2, lam, V)

    U = [[None] * 3 for _ in range(3)]
    for k in range(3):
        w = [(H[i][0] * V[0][k] + H[i][1] * V[1][k]) + H[i][2] * V[2][k]
             for i in range(3)]
        inv = jax.lax.rsqrt(
            jnp.maximum(w[0] * w[0] + w[1] * w[1] + w[2] * w[2], 1e-30))
        for i in range(3):
            U[i][k] = w[i] * inv

    det_h = (H[0][0] * (H[1][1] * H[2][2] - H[1][2] * H[2][1])
             - H[0][1] * (H[1][0] * H[2][2] - H[1][2] * H[2][0])
             + H[0][2] * (H[1][0] * H[2][1] - H[1][1] * H[2][0]))
    d = jnp.where(det_h >= 0, jnp.float32(1.0), jnp.float32(-1.0))

    Vb = [[_bf(V[i][k]) for k in range(3)] for i in range(3)]
    Ub = [[_bf(U[i][k]) for k in range(3)] for i in range(3)]
    R = [[(Vb[i][0] * Ub[l][0] + Vb[i][1] * Ub[l][1])
          + (d * Vb[i][2]) * Ub[l][2]
          for l in range(3)] for i in range(3)]
    t = [mu_t[i] - ((_bf(R[i][0]) * _bf(mu_s[0]) + _bf(R[i][1]) * _bf(mu_s[1]))
                    + _bf(R[i][2]) * _bf(mu_s[2]))
         for i in range(3)]
    return R, t


def _icp_body(psrc_ref, ptgtT_ref, out_ref, temppc_ref):
    psrc = psrc_ref[0]
    ptgtT = ptgtT_ref[0]
    t0 = ptgtT[0:1, :]
    t1 = ptgtT[1:2, :]
    t2 = ptgtT[2:3, :]
    t0b = _bf(t0)
    t1b = _bf(t1)
    t2b = _bf(t2)
    tgt2 = (t0 * t0 + t1 * t1) + t2 * t2
    col_iota = jax.lax.broadcasted_iota(jnp.int32, (_TILE, _M), 1)
    inv_n = jnp.float32(1.0 / _N)
    temppc_ref[...] = psrc

    def step(_, carry_dummy):
        temppc = temppc_ref[...]
        sum_s = [jnp.sum(temppc[:, i:i + 1]) for i in range(3)]
        mu_s = [v * inv_n for v in sum_s]

        def tile_body(i, carry):
            colcnt, acc = carry
            s_tile = temppc_ref[pl.ds(i * _TILE, _TILE), :]
            s0 = s_tile[:, 0:1]
            s1 = s_tile[:, 1:2]
            s2 = s_tile[:, 2:3]
            cross = (_bf(s0) * t0b + _bf(s1) * t1b) + _bf(s2) * t2b
            src2 = (s0 * s0 + s1 * s1) + s2 * s2
            score = jnp.maximum((src2 - 2.0 * cross) + tgt2, 0.0)
            rowmin = jnp.min(score, axis=1, keepdims=True)
            idx = jnp.min(jnp.where(score <= rowmin, col_iota, jnp.int32(_M)),
                          axis=1, keepdims=True)
            w_mask = (col_iota == idx).astype(jnp.float32)
            colcnt = colcnt + jnp.sum(w_mask, axis=0, keepdims=True)
            sc = jnp.concatenate(
                [_bf(s0 - mu_s[0]), _bf(s1 - mu_s[1]), _bf(s2 - mu_s[2])],
                axis=1)
            acc = acc + jax.lax.dot_general(
                sc, w_mask, (((0,), (0,)), ((), ())),
                preferred_element_type=jnp.float32)
            return colcnt, acc

        colcnt, acc = jax.lax.fori_loop(
            0, _N // _TILE, tile_body,
            (jnp.zeros((1, _M), jnp.float32), jnp.zeros((3, _M), jnp.float32)))

        s1sum = [jnp.sum(colcnt * ptgtT[j:j + 1, :]) for j in range(3)]
        mu_t = [v * inv_n for v in s1sum]
        tcb = [_bf(ptgtT[j:j + 1, :] - mu_t[j]) for j in range(3)]
        H = [[jnp.sum(acc[i:i + 1, :] * tcb[j]) for j in range(3)]
             for i in range(3)]
        R, t = _solve_rt(H, mu_s, mu_t)

        xb = _bf(temppc[:, 0:1])
        yb = _bf(temppc[:, 1:2])
        zb = _bf(temppc[:, 2:3])
        Rb = [[_bf(R[i][j]) for j in range(3)] for i in range(3)]
        temppc_ref[...] = jnp.concatenate(
            [((Rb[0][0] * xb + Rb[0][1] * yb) + Rb[0][2] * zb) + t[0],
             ((Rb[1][0] * xb + Rb[1][1] * yb) + Rb[1][2] * zb) + t[1],
             ((Rb[2][0] * xb + Rb[2][1] * yb) + Rb[2][2] * zb) + t[2]],
            axis=1)
        return carry_dummy

    jax.lax.fori_loop(0, _STEPS, step, jnp.int32(0))
    temppc = temppc_ref[...]

    sum_p = [jnp.sum(psrc[:, i:i + 1]) for i in range(3)]
    sum_c = [jnp.sum(temppc[:, i:i + 1]) for i in range(3)]
    mu_p = [v * inv_n for v in sum_p]
    mu_c = [v * inv_n for v in sum_c]
    pcb = [_bf(psrc[:, i:i + 1] - mu_p[i]) for i in range(3)]
    ccb = [_bf(temppc[:, j:j + 1] - mu_c[j]) for j in range(3)]
    Hf = [[jnp.sum(pcb[i] * ccb[j]) for j in range(3)] for i in range(3)]
    R, t = _solve_rt(Hf, mu_p, mu_c)

    row_i = jax.lax.broadcasted_iota(jnp.int32, (3, 4), 0)
    col_i = jax.lax.broadcasted_iota(jnp.int32, (3, 4), 1)
    outmat = jnp.zeros((3, 4), jnp.float32)
    vals = [[R[0][0], R[0][1], R[0][2], t[0]],
            [R[1][0], R[1][1], R[1][2], t[1]],
            [R[2][0], R[2][1], R[2][2], t[2]]]
    for i in range(3):
        for j in range(4):
            outmat = outmat + vals[i][j] * jnp.where(
                (row_i == i) & (col_i == j), jnp.float32(1.0), jnp.float32(0.0))
    out_ref[0] = outmat


@functools.partial(jax.jit, static_argnames=("interpret",))
def _icp_pallas(psrc, ptgt, interpret=False):
    B = psrc.shape[0]
    ptgtT = jnp.swapaxes(ptgt, -1, -2)
    return pl.pallas_call(
        _icp_body,
        grid=(B,),
        in_specs=[
            pl.BlockSpec((1, _N, 3), lambda b: (b, 0, 0)),
            pl.BlockSpec((1, 3, _M), lambda b: (b, 0, 0)),
        ],
        out_specs=pl.BlockSpec((1, 3, 4), lambda b: (b, 0, 0)),
        out_shape=jax.ShapeDtypeStruct((B, 3, 4), jnp.float32),
        scratch_shapes=[pltpu.VMEM((_N, 3), jnp.float32)],
        compiler_params=pltpu.CompilerParams(
            dimension_semantics=("parallel",)),
        interpret=interpret,
    )(psrc, ptgtT)


def kernel(psrc, ptgt):
    return _icp_pallas(psrc, ptgt)


# speedup vs baseline: 1.6450x; 1.6450x over previous
import functools

import jax
import jax.numpy as jnp
from jax.experimental import pallas as pl
from jax.experimental.pallas import tpu as pltpu

_N = 4096
_M = 4096
_TILE = 512
_STEPS = 10
_SWEEPS3 = 6


def _bf(x):
    return x.astype(jnp.bfloat16).astype(jnp.float32)


def _jacobi3(A):
    a = dict(A)
    V = [[jnp.float32(1.0) if i == j else jnp.float32(0.0) for j in range(3)]
         for i in range(3)]

    def get(i, j):
        return a[(i, j)] if i <= j else a[(j, i)]

    def put(i, j, v):
        a[(i, j) if i <= j else (j, i)] = v

    for _ in range(_SWEEPS3):
        for (p, q) in ((0, 1), (0, 2), (1, 2)):
            apq = get(p, q)
            app = get(p, p)
            aqq = get(q, q)
            small = jnp.abs(apq) < 1e-30
            apq_s = jnp.where(small, 1.0, apq)
            theta = (aqq - app) / (2.0 * apq_s)
            r = jnp.sqrt(theta * theta + 1.0)
            t = jnp.where(theta >= 0, 1.0 / (theta + r), -1.0 / (r - theta))
            t = jnp.where(small, 0.0, t)
            c = jax.lax.rsqrt(t * t + 1.0)
            s = t * c
            k = 3 - p - q
            akp = get(k, p)
            akq = get(k, q)
            put(k, p, c * akp - s * akq)
            put(k, q, s * akp + c * akq)
            put(p, p, app - t * apq)
            put(q, q, aqq + t * apq)
            put(p, q, jnp.float32(0.0))
            for kk in range(3):
                vkp = V[kk][p]
                vkq = V[kk][q]
                V[kk][p] = c * vkp - s * vkq
                V[kk][q] = s * vkp + c * vkq
    return [get(i, i) for i in range(3)], V


def _solve_rt(H, mu_s, mu_t):
    A = {}
    for i in range(3):
        for j in range(i, 3):
            A[(i, j)] = (H[0][i] * H[0][j] + H[1][i] * H[1][j]) \
                + H[2][i] * H[2][j]
    lam, V = _jacobi3(A)

    def cs(i, j, lam, V):
        sw = lam[j] > lam[i]
        li = jnp.where(sw, lam[j], lam[i])
        lj = jnp.where(sw, lam[i], lam[j])
        lam = list(lam)
        lam[i], lam[j] = li, lj
        V = [row[:] for row in V]
        for r in range(3):
            vi = jnp.where(sw, V[r][j], V[r][i])
            vj = jnp.where(sw, V[r][i], V[r][j])
            V[r][i], V[r][j] = vi, vj
        return lam, V

    lam, V = cs(0, 1, lam, V)
    lam, V = cs(0, 2, lam, V)
    lam, V = cs(1, 2, lam, V)

    U = [[None] * 3 for _ in range(3)]
    for k in range(3):
        w = [(H[i][0] * V[0][k] + H[i][1] * V[1][k]) + H[i][2] * V[2][k]
             for i in range(3)]
        inv = jax.lax.rsqrt(
            jnp.maximum(w[0] * w[0] + w[1] * w[1] + w[2] * w[2], 1e-30))
        for i in range(3):
            U[i][k] = w[i] * inv

    det_h = (H[0][0] * (H[1][1] * H[2][2] - H[1][2] * H[2][1])
             - H[0][1] * (H[1][0] * H[2][2] - H[1][2] * H[2][0])
             + H[0][2] * (H[1][0] * H[2][1] - H[1][1] * H[2][0]))
    d = jnp.where(det_h >= 0, jnp.float32(1.0), jnp.float32(-1.0))

    Vb = [[_bf(V[i][k]) for k in range(3)] for i in range(3)]
    Ub = [[_bf(U[i][k]) for k in range(3)] for i in range(3)]
    R = [[(Vb[i][0] * Ub[l][0] + Vb[i][1] * Ub[l][1])
          + (d * Vb[i][2]) * Ub[l][2]
          for l in range(3)] for i in range(3)]
    t = [mu_t[i] - ((_bf(R[i][0]) * _bf(mu_s[0]) + _bf(R[i][1]) * _bf(mu_s[1]))
                    + _bf(R[i][2]) * _bf(mu_s[2]))
         for i in range(3)]
    return R, t


def _icp_body(psrcT_ref, ptgtT_ref, out_ref, temppc_ref):
    psrcT = psrcT_ref[0]
    ptgtT = ptgtT_ref[0]
    t0 = ptgtT[0:1, :]
    t1 = ptgtT[1:2, :]
    t2 = ptgtT[2:3, :]
    tgt_bf = ptgtT.astype(jnp.bfloat16)
    tgt2 = (t0 * t0 + t1 * t1) + t2 * t2
    col_iota = jax.lax.broadcasted_iota(jnp.int32, (_TILE, _M), 1)
    ones_row = jnp.ones((1, _TILE), jnp.bfloat16)
    inv_n = jnp.float32(1.0 / _N)
    temppc_ref[...] = psrcT

    def step(_, carry_dummy):
        sum_s = [jnp.sum(temppc_ref[i:i + 1, :]) for i in range(3)]
        mu_s = [v * inv_n for v in sum_s]

        def tile_body(i, acc4):
            s_tile = temppc_ref[:, pl.ds(i * _TILE, _TILE)]
            s0 = s_tile[0:1, :]
            s1 = s_tile[1:2, :]
            s2 = s_tile[2:3, :]
            cross = jax.lax.dot_general(
                s_tile.astype(jnp.bfloat16), tgt_bf,
                (((0,), (0,)), ((), ())),
                preferred_element_type=jnp.float32)
            src2 = jnp.reshape((s0 * s0 + s1 * s1) + s2 * s2,
                               (_TILE, 1))
            score = jnp.maximum((src2 - 2.0 * cross) + tgt2, 0.0)
            rowmin = jnp.min(score, axis=1, keepdims=True)
            idx = jnp.min(jnp.where(score <= rowmin, col_iota, jnp.int32(_M)),
                          axis=1, keepdims=True)
            w_mask = (col_iota == idx).astype(jnp.bfloat16)
            sc4 = jnp.concatenate(
                [_bf(s0 - mu_s[0]).astype(jnp.bfloat16),
                 _bf(s1 - mu_s[1]).astype(jnp.bfloat16),
                 _bf(s2 - mu_s[2]).astype(jnp.bfloat16),
                 ones_row], axis=0)
            return acc4 + jax.lax.dot_general(
                sc4, w_mask, (((1,), (0,)), ((), ())),
                preferred_element_type=jnp.float32)

        acc4 = jax.lax.fori_loop(
            0, _N // _TILE, tile_body, jnp.zeros((4, _M), jnp.float32))

        colcnt = acc4[3:4, :]
        s1sum = [jnp.sum(colcnt * ptgtT[j:j + 1, :]) for j in range(3)]
        mu_t = [v * inv_n for v in s1sum]
        tcb = [_bf(ptgtT[j:j + 1, :] - mu_t[j]) for j in range(3)]
        H = [[jnp.sum(acc4[i:i + 1, :] * tcb[j]) for j in range(3)]
             for i in range(3)]
        R, t = _solve_rt(H, mu_s, mu_t)

        xb = _bf(temppc_ref[0:1, :])
        yb = _bf(temppc_ref[1:2, :])
        zb = _bf(temppc_ref[2:3, :])
        Rb = [[_bf(R[i][j]) for j in range(3)] for i in range(3)]
        temppc_ref[...] = jnp.concatenate(
            [((Rb[0][0] * xb + Rb[0][1] * yb) + Rb[0][2] * zb) + t[0],
             ((Rb[1][0] * xb + Rb[1][1] * yb) + Rb[1][2] * zb) + t[1],
             ((Rb[2][0] * xb + Rb[2][1] * yb) + Rb[2][2] * zb) + t[2]],
            axis=0)
        return carry_dummy

    jax.lax.fori_loop(0, _STEPS, step, jnp.int32(0))

    sum_p = [jnp.sum(psrcT[i:i + 1, :]) for i in range(3)]
    sum_c = [jnp.sum(temppc_ref[i:i + 1, :]) for i in range(3)]
    mu_p = [v * inv_n for v in sum_p]
    mu_c = [v * inv_n for v in sum_c]
    pcb = [_bf(psrcT[i:i + 1, :] - mu_p[i]) for i in range(3)]
    ccb = [_bf(temppc_ref[j:j + 1, :] - mu_c[j]) for j in range(3)]
    Hf = [[jnp.sum(pcb[i] * ccb[j]) for j in range(3)] for i in range(3)]
    R, t = _solve_rt(Hf, mu_p, mu_c)

    row_i = jax.lax.broadcasted_iota(jnp.int32, (3, 4), 0)
    col_i = jax.lax.broadcasted_iota(jnp.int32, (3, 4), 1)
    outmat = jnp.zeros((3, 4), jnp.float32)
    vals = [[R[0][0], R[0][1], R[0][2], t[0]],
            [R[1][0], R[1][1], R[1][2], t[1]],
            [R[2][0], R[2][1], R[2][2], t[2]]]
    for i in range(3):
        for j in range(4):
            outmat = outmat + vals[i][j] * jnp.where(
                (row_i == i) & (col_i == j), jnp.float32(1.0), jnp.float32(0.0))
    out_ref[0] = outmat


@functools.partial(jax.jit, static_argnames=("interpret",))
def _icp_pallas(psrc, ptgt, interpret=False):
    B = psrc.shape[0]
    psrcT = jnp.swapaxes(psrc, -1, -2)
    ptgtT = jnp.swapaxes(ptgt, -1, -2)
    return pl.pallas_call(
        _icp_body,
        grid=(B,),
        in_specs=[
            pl.BlockSpec((1, 3, _N), lambda b: (b, 0, 0)),
            pl.BlockSpec((1, 3, _M), lambda b: (b, 0, 0)),
        ],
        out_specs=pl.BlockSpec((1, 3, 4), lambda b: (b, 0, 0)),
        out_shape=jax.ShapeDtypeStruct((B, 3, 4), jnp.float32),
        scratch_shapes=[pltpu.VMEM((3, _N), jnp.float32)],
        compiler_params=pltpu.CompilerParams(
            dimension_semantics=("parallel",)),
        interpret=interpret,
    )(psrcT, ptgtT)


def kernel(psrc, ptgt):
    return _icp_pallas(psrc, ptgt)


# speedup vs baseline: 1.8309x; 1.1130x over previous
import functools

import jax
import jax.numpy as jnp
from jax.experimental import pallas as pl
from jax.experimental.pallas import tpu as pltpu

_N = 4096
_M = 4096
_TILE = 512
_STEPS = 10
_SWEEPS3 = 6


def _bf(x):
    return x.astype(jnp.bfloat16).astype(jnp.float32)


def _jacobi3(A):
    a = dict(A)
    V = [[jnp.float32(1.0) if i == j else jnp.float32(0.0) for j in range(3)]
         for i in range(3)]

    def get(i, j):
        return a[(i, j)] if i <= j else a[(j, i)]

    def put(i, j, v):
        a[(i, j) if i <= j else (j, i)] = v

    for _ in range(_SWEEPS3):
        for (p, q) in ((0, 1), (0, 2), (1, 2)):
            apq = get(p, q)
            app = get(p, p)
            aqq = get(q, q)
            small = jnp.abs(apq) < 1e-30
            apq_s = jnp.where(small, 1.0, apq)
            theta = (aqq - app) / (2.0 * apq_s)
            r = jnp.sqrt(theta * theta + 1.0)
            t = jnp.where(theta >= 0, 1.0 / (theta + r), -1.0 / (r - theta))
            t = jnp.where(small, 0.0, t)
            c = jax.lax.rsqrt(t * t + 1.0)
            s = t * c
            k = 3 - p - q
            akp = get(k, p)
            akq = get(k, q)
            put(k, p, c * akp - s * akq)
            put(k, q, s * akp + c * akq)
            put(p, p, app - t * apq)
            put(q, q, aqq + t * apq)
            put(p, q, jnp.float32(0.0))
            for kk in range(3):
                vkp = V[kk][p]
                vkq = V[kk][q]
                V[kk][p] = c * vkp - s * vkq
                V[kk][q] = s * vkp + c * vkq
    return [get(i, i) for i in range(3)], V


def _solve_rt(H, mu_s, mu_t):
    A = {}
    for i in range(3):
        for j in range(i, 3):
            A[(i, j)] = (H[0][i] * H[0][j] + H[1][i] * H[1][j]) \
                + H[2][i] * H[2][j]
    lam, V = _jacobi3(A)

    def cs(i, j, lam, V):
        sw = lam[j] > lam[i]
        li = jnp.where(sw, lam[j], lam[i])
        lj = jnp.where(sw, lam[i], lam[j])
        lam = list(lam)
        lam[i], lam[j] = li, lj
        V = [row[:] for row in V]
        for r in range(3):
            vi = jnp.where(sw, V[r][j], V[r][i])
            vj = jnp.where(sw, V[r][i], V[r][j])
            V[r][i], V[r][j] = vi, vj
        return lam, V

    lam, V = cs(0, 1, lam, V)
    lam, V = cs(0, 2, lam, V)
    lam, V = cs(1, 2, lam, V)

    U = [[None] * 3 for _ in range(3)]
    for k in range(3):
        w = [(H[i][0] * V[0][k] + H[i][1] * V[1][k]) + H[i][2] * V[2][k]
             for i in range(3)]
        inv = jax.lax.rsqrt(
            jnp.maximum(w[0] * w[0] + w[1] * w[1] + w[2] * w[2], 1e-30))
        for i in range(3):
            U[i][k] = w[i] * inv

    det_h = (H[0][0] * (H[1][1] * H[2][2] - H[1][2] * H[2][1])
             - H[0][1] * (H[1][0] * H[2][2] - H[1][2] * H[2][0])
             + H[0][2] * (H[1][0] * H[2][1] - H[1][1] * H[2][0]))
    d = jnp.where(det_h >= 0, jnp.float32(1.0), jnp.float32(-1.0))

    Vb = [[_bf(V[i][k]) for k in range(3)] for i in range(3)]
    Ub = [[_bf(U[i][k]) for k in range(3)] for i in range(3)]
    R = [[(Vb[i][0] * Ub[l][0] + Vb[i][1] * Ub[l][1])
          + (d * Vb[i][2]) * Ub[l][2]
          for l in range(3)] for i in range(3)]
    t = [mu_t[i] - ((_bf(R[i][0]) * _bf(mu_s[0]) + _bf(R[i][1]) * _bf(mu_s[1]))
                    + _bf(R[i][2]) * _bf(mu_s[2]))
         for i in range(3)]
    return R, t


def _icp_body(psrcT_ref, ptgtT_ref, out_ref, temppc_ref):
    psrcT = psrcT_ref[0]
    ptgtT = ptgtT_ref[0]
    t0 = ptgtT[0:1, :]
    t1 = ptgtT[1:2, :]
    t2 = ptgtT[2:3, :]
    tgt_bf = ptgtT.astype(jnp.bfloat16)
    tgt2 = (t0 * t0 + t1 * t1) + t2 * t2
    iota_row = jax.lax.broadcasted_iota(
        jnp.int32, (1, _M), 1).astype(jnp.float32)
    ones_row = jnp.ones((1, _N), jnp.bfloat16)
    inv_n = jnp.float32(1.0 / _N)
    temppc_ref[...] = psrcT

    def step(_, carry_dummy):
        x = temppc_ref[0:1, :]
        y = temppc_ref[1:2, :]
        z = temppc_ref[2:3, :]
        sum_s = [jnp.sum(x), jnp.sum(y), jnp.sum(z)]
        mu_s = [v * inv_n for v in sum_s]
        s_bf = temppc_ref[...].astype(jnp.bfloat16)
        sc4 = jnp.concatenate(
            [(x - mu_s[0]).astype(jnp.bfloat16),
             (y - mu_s[1]).astype(jnp.bfloat16),
             (z - mu_s[2]).astype(jnp.bfloat16),
             ones_row], axis=0)
        src2_row = (x * x + y * y) + z * z

        acc4 = jnp.zeros((4, _M), jnp.float32)
        for i in range(_N // _TILE):
            lo, hi = i * _TILE, (i + 1) * _TILE
            cross = jax.lax.dot_general(
                s_bf[:, lo:hi], tgt_bf,
                (((0,), (0,)), ((), ())),
                preferred_element_type=jnp.float32)
            src2 = jnp.reshape(src2_row[:, lo:hi], (_TILE, 1))
            score = jnp.maximum((src2 - 2.0 * cross) + tgt2, 0.0)
            rowmin = jnp.min(score, axis=1, keepdims=True)
            idx = jnp.min(jnp.where(score <= rowmin, iota_row,
                                    jnp.float32(_M)),
                          axis=1, keepdims=True)
            w_mask = (iota_row == idx).astype(jnp.bfloat16)
            acc4 = acc4 + jax.lax.dot_general(
                sc4[:, lo:hi], w_mask,
                (((1,), (0,)), ((), ())),
                preferred_element_type=jnp.float32)

        colcnt = acc4[3:4, :]
        s1sum = [jnp.sum(colcnt * ptgtT[j:j + 1, :]) for j in range(3)]
        mu_t = [v * inv_n for v in s1sum]
        tcb = [_bf(ptgtT[j:j + 1, :] - mu_t[j]) for j in range(3)]
        H = [[jnp.sum(acc4[i:i + 1, :] * tcb[j]) for j in range(3)]
             for i in range(3)]
        R, t = _solve_rt(H, mu_s, mu_t)

        xb = _bf(temppc_ref[0:1, :])
        yb = _bf(temppc_ref[1:2, :])
        zb = _bf(temppc_ref[2:3, :])
        Rb = [[_bf(R[i][j]) for j in range(3)] for i in range(3)]
        temppc_ref[...] = jnp.concatenate(
            [((Rb[0][0] * xb + Rb[0][1] * yb) + Rb[0][2] * zb) + t[0],
             ((Rb[1][0] * xb + Rb[1][1] * yb) + Rb[1][2] * zb) + t[1],
             ((Rb[2][0] * xb + Rb[2][1] * yb) + Rb[2][2] * zb) + t[2]],
            axis=0)
        return carry_dummy

    jax.lax.fori_loop(0, _STEPS, step, jnp.int32(0))

    sum_p = [jnp.sum(psrcT[i:i + 1, :]) for i in range(3)]
    sum_c = [jnp.sum(temppc_ref[i:i + 1, :]) for i in range(3)]
    mu_p = [v * inv_n for v in sum_p]
    mu_c = [v * inv_n for v in sum_c]
    pcb = [_bf(psrcT[i:i + 1, :] - mu_p[i]) for i in range(3)]
    ccb = [_bf(temppc_ref[j:j + 1, :] - mu_c[j]) for j in range(3)]
    Hf = [[jnp.sum(pcb[i] * ccb[j]) for j in range(3)] for i in range(3)]
    R, t = _solve_rt(Hf, mu_p, mu_c)

    row_i = jax.lax.broadcasted_iota(jnp.int32, (3, 4), 0)
    col_i = jax.lax.broadcasted_iota(jnp.int32, (3, 4), 1)
    outmat = jnp.zeros((3, 4), jnp.float32)
    vals = [[R[0][0], R[0][1], R[0][2], t[0]],
            [R[1][0], R[1][1], R[1][2], t[1]],
            [R[2][0], R[2][1], R[2][2], t[2]]]
    for i in range(3):
        for j in range(4):
            outmat = outmat + vals[i][j] * jnp.where(
                (row_i == i) & (col_i == j), jnp.float32(1.0), jnp.float32(0.0))
    out_ref[0] = outmat


@functools.partial(jax.jit, static_argnames=("interpret",))
def _icp_pallas(psrc, ptgt, interpret=False):
    B = psrc.shape[0]
    psrcT = jnp.swapaxes(psrc, -1, -2)
    ptgtT = jnp.swapaxes(ptgt, -1, -2)
    return pl.pallas_call(
        _icp_body,
        grid=(B,),
        in_specs=[
            pl.BlockSpec((1, 3, _N), lambda b: (b, 0, 0)),
            pl.BlockSpec((1, 3, _M), lambda b: (b, 0, 0)),
        ],
        out_specs=pl.BlockSpec((1, 3, 4), lambda b: (b, 0, 0)),
        out_shape=jax.ShapeDtypeStruct((B, 3, 4), jnp.float32),
        scratch_shapes=[pltpu.VMEM((3, _N), jnp.float32)],
        compiler_params=pltpu.CompilerParams(
            dimension_semantics=("parallel",)),
        interpret=interpret,
    )(psrcT, ptgtT)


def kernel(psrc, ptgt):
    return _icp_pallas(psrc, ptgt)


# speedup vs baseline: 1.8676x; 1.0200x over previous
import functools

import jax
import jax.numpy as jnp
from jax.experimental import pallas as pl
from jax.experimental.pallas import tpu as pltpu

_N = 4096
_M = 4096
_TILE = 1024
_STEPS = 10
_SWEEPS3 = 6


def _bf(x):
    return x.astype(jnp.bfloat16).astype(jnp.float32)


def _jacobi3(A):
    a = dict(A)
    V = [[jnp.float32(1.0) if i == j else jnp.float32(0.0) for j in range(3)]
         for i in range(3)]

    def get(i, j):
        return a[(i, j)] if i <= j else a[(j, i)]

    def put(i, j, v):
        a[(i, j) if i <= j else (j, i)] = v

    for _ in range(_SWEEPS3):
        for (p, q) in ((0, 1), (0, 2), (1, 2)):
            apq = get(p, q)
            app = get(p, p)
            aqq = get(q, q)
            small = jnp.abs(apq) < 1e-30
            apq_s = jnp.where(small, 1.0, apq)
            theta = (aqq - app) / (2.0 * apq_s)
            r = jnp.sqrt(theta * theta + 1.0)
            t = jnp.where(theta >= 0, 1.0 / (theta + r), -1.0 / (r - theta))
            t = jnp.where(small, 0.0, t)
            c = jax.lax.rsqrt(t * t + 1.0)
            s = t * c
            k = 3 - p - q
            akp = get(k, p)
            akq = get(k, q)
            put(k, p, c * akp - s * akq)
            put(k, q, s * akp + c * akq)
            put(p, p, app - t * apq)
            put(q, q, aqq + t * apq)
            put(p, q, jnp.float32(0.0))
            for kk in range(3):
                vkp = V[kk][p]
                vkq = V[kk][q]
                V[kk][p] = c * vkp - s * vkq
                V[kk][q] = s * vkp + c * vkq
    return [get(i, i) for i in range(3)], V


def _solve_rt(H, mu_s, mu_t):
    A = {}
    for i in range(3):
        for j in range(i, 3):
            A[(i, j)] = (H[0][i] * H[0][j] + H[1][i] * H[1][j]) \
                + H[2][i] * H[2][j]
    lam, V = _jacobi3(A)

    def cs(i, j, lam, V):
        sw = lam[j] > lam[i]
        li = jnp.where(sw, lam[j], lam[i])
        lj = jnp.where(sw, lam[i], lam[j])
        lam = list(lam)
        lam[i], lam[j] = li, lj
        V = [row[:] for row in V]
        for r in range(3):
            vi = jnp.where(sw, V[r][j], V[r][i])
            vj = jnp.where(sw, V[r][i], V[r][j])
            V[r][i], V[r][j] = vi, vj
        return lam, V

    lam, V = cs(0, 1, lam, V)
    lam, V = cs(0, 2, lam, V)
    lam, V = cs(1, 2, lam, V)

    U = [[None] * 3 for _ in range(3)]
    for k in range(3):
        w = [(H[i][0] * V[0][k] + H[i][1] * V[1][k]) + H[i][2] * V[2][k]
             for i in range(3)]
        inv = jax.lax.rsqrt(
            jnp.maximum(w[0] * w[0] + w[1] * w[1] + w[2] * w[2], 1e-30))
        for i in range(3):
            U[i][k] = w[i] * inv

    det_h = (H[0][0] * (H[1][1] * H[2][2] - H[1][2] * H[2][1])
             - H[0][1] * (H[1][0] * H[2][2] - H[1][2] * H[2][0])
             + H[0][2] * (H[1][0] * H[2][1] - H[1][1] * H[2][0]))
    d = jnp.where(det_h >= 0, jnp.float32(1.0), jnp.float32(-1.0))

    Vb = [[_bf(V[i][k]) for k in range(3)] for i in range(3)]
    Ub = [[_bf(U[i][k]) for k in range(3)] for i in range(3)]
    R = [[(Vb[i][0] * Ub[l][0] + Vb[i][1] * Ub[l][1])
          + (d * Vb[i][2]) * Ub[l][2]
          for l in range(3)] for i in range(3)]
    t = [mu_t[i] - ((_bf(R[i][0]) * _bf(mu_s[0]) + _bf(R[i][1]) * _bf(mu_s[1]))
                    + _bf(R[i][2]) * _bf(mu_s[2]))
         for i in range(3)]
    return R, t


def _icp_body(psrcT_ref, ptgtT_ref, out_ref, temppc_ref):
    psrcT = psrcT_ref[0]
    ptgtT = ptgtT_ref[0]
    t0 = ptgtT[0:1, :]
    t1 = ptgtT[1:2, :]
    t2 = ptgtT[2:3, :]
    tgt_bf = ptgtT.astype(jnp.bfloat16)
    tgt2 = (t0 * t0 + t1 * t1) + t2 * t2
    iota_row = jax.lax.broadcasted_iota(
        jnp.int32, (1, _M), 1).astype(jnp.float32)
    ones_row = jnp.ones((1, _N), jnp.bfloat16)
    inv_n = jnp.float32(1.0 / _N)
    temppc_ref[...] = psrcT

    def step(_, carry_dummy):
        x = temppc_ref[0:1, :]
        y = temppc_ref[1:2, :]
        z = temppc_ref[2:3, :]
        sum_s = [jnp.sum(x), jnp.sum(y), jnp.sum(z)]
        mu_s = [v * inv_n for v in sum_s]
        s_bf = temppc_ref[...].astype(jnp.bfloat16)
        sc4 = jnp.concatenate(
            [(x - mu_s[0]).astype(jnp.bfloat16),
             (y - mu_s[1]).astype(jnp.bfloat16),
             (z - mu_s[2]).astype(jnp.bfloat16),
             ones_row], axis=0)
        src2_row = (x * x + y * y) + z * z

        acc4 = jnp.zeros((4, _M), jnp.float32)
        for i in range(_N // _TILE):
            lo, hi = i * _TILE, (i + 1) * _TILE
            cross = jax.lax.dot_general(
                s_bf[:, lo:hi], tgt_bf,
                (((0,), (0,)), ((), ())),
                preferred_element_type=jnp.float32)
            src2 = jnp.reshape(src2_row[:, lo:hi], (_TILE, 1))
            score = jnp.maximum((src2 - 2.0 * cross) + tgt2, 0.0)
            rowmin = jnp.min(score, axis=1, keepdims=True)
            idx = jnp.min(jnp.where(score <= rowmin, iota_row,
                                    jnp.float32(_M)),
                          axis=1, keepdims=True)
            w_mask = (iota_row == idx).astype(jnp.bfloat16)
            acc4 = acc4 + jax.lax.dot_general(
                sc4[:, lo:hi], w_mask,
                (((1,), (0,)), ((), ())),
                preferred_element_type=jnp.float32)

        colcnt = acc4[3:4, :]
        s1sum = [jnp.sum(colcnt * ptgtT[j:j + 1, :]) for j in range(3)]
        mu_t = [v * inv_n for v in s1sum]
        tcb = [_bf(ptgtT[j:j + 1, :] - mu_t[j]) for j in range(3)]
        H = [[jnp.sum(acc4[i:i + 1, :] * tcb[j]) for j in range(3)]
             for i in range(3)]
        R, t = _solve_rt(H, mu_s, mu_t)

        xb = _bf(temppc_ref[0:1, :])
        yb = _bf(temppc_ref[1:2, :])
        zb = _bf(temppc_ref[2:3, :])
        Rb = [[_bf(R[i][j]) for j in range(3)] for i in range(3)]
        temppc_ref[...] = jnp.concatenate(
            [((Rb[0][0] * xb + Rb[0][1] * yb) + Rb[0][2] * zb) + t[0],
             ((Rb[1][0] * xb + Rb[1][1] * yb) + Rb[1][2] * zb) + t[1],
             ((Rb[2][0] * xb + Rb[2][1] * yb) + Rb[2][2] * zb) + t[2]],
            axis=0)
        return carry_dummy

    jax.lax.fori_loop(0, _STEPS, step, jnp.int32(0))

    sum_p = [jnp.sum(psrcT[i:i + 1, :]) for i in range(3)]
    sum_c = [jnp.sum(temppc_ref[i:i + 1, :]) for i in range(3)]
    mu_p = [v * inv_n for v in sum_p]
    mu_c = [v * inv_n for v in sum_c]
    pcb = [_bf(psrcT[i:i + 1, :] - mu_p[i]) for i in range(3)]
    ccb = [_bf(temppc_ref[j:j + 1, :] - mu_c[j]) for j in range(3)]
    Hf = [[jnp.sum(pcb[i] * ccb[j]) for j in range(3)] for i in range(3)]
    R, t = _solve_rt(Hf, mu_p, mu_c)

    row_i = jax.lax.broadcasted_iota(jnp.int32, (3, 4), 0)
    col_i = jax.lax.broadcasted_iota(jnp.int32, (3, 4), 1)
    outmat = jnp.zeros((3, 4), jnp.float32)
    vals = [[R[0][0], R[0][1], R[0][2], t[0]],
            [R[1][0], R[1][1], R[1][2], t[1]],
            [R[2][0], R[2][1], R[2][2], t[2]]]
    for i in range(3):
        for j in range(4):
            outmat = outmat + vals[i][j] * jnp.where(
                (row_i == i) & (col_i == j), jnp.float32(1.0), jnp.float32(0.0))
    out_ref[0] = outmat


@functools.partial(jax.jit, static_argnames=("interpret",))
def _icp_pallas(psrc, ptgt, interpret=False):
    B = psrc.shape[0]
    psrcT = jnp.swapaxes(psrc, -1, -2)
    ptgtT = jnp.swapaxes(ptgt, -1, -2)
    return pl.pallas_call(
        _icp_body,
        grid=(B,),
        in_specs=[
            pl.BlockSpec((1, 3, _N), lambda b: (b, 0, 0)),
            pl.BlockSpec((1, 3, _M), lambda b: (b, 0, 0)),
        ],
        out_specs=pl.BlockSpec((1, 3, 4), lambda b: (b, 0, 0)),
        out_shape=jax.ShapeDtypeStruct((B, 3, 4), jnp.float32),
        scratch_shapes=[pltpu.VMEM((3, _N), jnp.float32)],
        compiler_params=pltpu.CompilerParams(
            dimension_semantics=("parallel",)),
        interpret=interpret,
    )(psrcT, ptgtT)


def kernel(psrc, ptgt):
    return _icp_pallas(psrc, ptgt)


# speedup vs baseline: 2.1726x; 1.1633x over previous
import functools

import jax
import jax.numpy as jnp
from jax.experimental import pallas as pl
from jax.experimental.pallas import tpu as pltpu

_N = 4096
_M = 4096
_TILE = 1024
_STEPS = 10
_SWEEPS3 = 6


def _bf(x):
    return x.astype(jnp.bfloat16).astype(jnp.float32)


def _jacobi3(A):
    a = dict(A)
    V = [[jnp.float32(1.0) if i == j else jnp.float32(0.0) for j in range(3)]
         for i in range(3)]

    def get(i, j):
        return a[(i, j)] if i <= j else a[(j, i)]

    def put(i, j, v):
        a[(i, j) if i <= j else (j, i)] = v

    for _ in range(_SWEEPS3):
        for (p, q) in ((0, 1), (0, 2), (1, 2)):
            apq = get(p, q)
            app = get(p, p)
            aqq = get(q, q)
            small = jnp.abs(apq) < 1e-30
            apq_s = jnp.where(small, 1.0, apq)
            theta = (aqq - app) / (2.0 * apq_s)
            r = jnp.sqrt(theta * theta + 1.0)
            t = jnp.where(theta >= 0, 1.0 / (theta + r), -1.0 / (r - theta))
            t = jnp.where(small, 0.0, t)
            c = jax.lax.rsqrt(t * t + 1.0)
            s = t * c
            k = 3 - p - q
            akp = get(k, p)
            akq = get(k, q)
            put(k, p, c * akp - s * akq)
            put(k, q, s * akp + c * akq)
            put(p, p, app - t * apq)
            put(q, q, aqq + t * apq)
            put(p, q, jnp.float32(0.0))
            for kk in range(3):
                vkp = V[kk][p]
                vkq = V[kk][q]
                V[kk][p] = c * vkp - s * vkq
                V[kk][q] = s * vkp + c * vkq
    return [get(i, i) for i in range(3)], V


def _solve_rt(H, mu_s, mu_t):
    A = {}
    for i in range(3):
        for j in range(i, 3):
            A[(i, j)] = (H[0][i] * H[0][j] + H[1][i] * H[1][j]) \
                + H[2][i] * H[2][j]
    lam, V = _jacobi3(A)

    def cs(i, j, lam, V):
        sw = lam[j] > lam[i]
        li = jnp.where(sw, lam[j], lam[i])
        lj = jnp.where(sw, lam[i], lam[j])
        lam = list(lam)
        lam[i], lam[j] = li, lj
        V = [row[:] for row in V]
        for r in range(3):
            vi = jnp.where(sw, V[r][j], V[r][i])
            vj = jnp.where(sw, V[r][i], V[r][j])
            V[r][i], V[r][j] = vi, vj
        return lam, V

    lam, V = cs(0, 1, lam, V)
    lam, V = cs(0, 2, lam, V)
    lam, V = cs(1, 2, lam, V)

    U = [[None] * 3 for _ in range(3)]
    for k in range(3):
        w = [(H[i][0] * V[0][k] + H[i][1] * V[1][k]) + H[i][2] * V[2][k]
             for i in range(3)]
        inv = jax.lax.rsqrt(
            jnp.maximum(w[0] * w[0] + w[1] * w[1] + w[2] * w[2], 1e-30))
        for i in range(3):
            U[i][k] = w[i] * inv

    det_h = (H[0][0] * (H[1][1] * H[2][2] - H[1][2] * H[2][1])
             - H[0][1] * (H[1][0] * H[2][2] - H[1][2] * H[2][0])
             + H[0][2] * (H[1][0] * H[2][1] - H[1][1] * H[2][0]))
    d = jnp.where(det_h >= 0, jnp.float32(1.0), jnp.float32(-1.0))

    Vb = [[_bf(V[i][k]) for k in range(3)] for i in range(3)]
    Ub = [[_bf(U[i][k]) for k in range(3)] for i in range(3)]
    R = [[(Vb[i][0] * Ub[l][0] + Vb[i][1] * Ub[l][1])
          + (d * Vb[i][2]) * Ub[l][2]
          for l in range(3)] for i in range(3)]
    t = [mu_t[i] - ((_bf(R[i][0]) * _bf(mu_s[0]) + _bf(R[i][1]) * _bf(mu_s[1]))
                    + _bf(R[i][2]) * _bf(mu_s[2]))
         for i in range(3)]
    return R, t


def _icp_body(psrcT_ref, ptgtT_ref, out_ref, temppc_ref):
    psrcT = psrcT_ref[0]
    ptgtT = ptgtT_ref[0]
    t0 = ptgtT[0:1, :]
    t1 = ptgtT[1:2, :]
    t2 = ptgtT[2:3, :]
    tgt_bf = ptgtT.astype(jnp.bfloat16)
    tgt2 = (t0 * t0 + t1 * t1) + t2 * t2
    iota_row = jax.lax.broadcasted_iota(
        jnp.int32, (1, _M), 1).astype(jnp.float32)
    ones_row = jnp.ones((1, _N), jnp.bfloat16)
    inv_n = jnp.float32(1.0 / _N)
    temppc_ref[...] = psrcT

    def step(_, carry_dummy):
        x = temppc_ref[0:1, :]
        y = temppc_ref[1:2, :]
        z = temppc_ref[2:3, :]
        sum_s = [jnp.sum(x), jnp.sum(y), jnp.sum(z)]
        mu_s = [v * inv_n for v in sum_s]
        sm2_bf = (temppc_ref[...] * -2.0).astype(jnp.bfloat16)
        sc4 = jnp.concatenate(
            [(x - mu_s[0]).astype(jnp.bfloat16),
             (y - mu_s[1]).astype(jnp.bfloat16),
             (z - mu_s[2]).astype(jnp.bfloat16),
             ones_row], axis=0)
        src2_row = (x * x + y * y) + z * z

        acc4 = jnp.zeros((4, _M), jnp.float32)
        for i in range(_N // _TILE):
            lo, hi = i * _TILE, (i + 1) * _TILE
            crossm2 = jax.lax.dot_general(
                sm2_bf[:, lo:hi], tgt_bf,
                (((0,), (0,)), ((), ())),
                preferred_element_type=jnp.float32)
            src2 = jnp.reshape(src2_row[:, lo:hi], (_TILE, 1))
            score = (src2 + crossm2) + tgt2
            rowmin = jnp.maximum(jnp.min(score, axis=1, keepdims=True), 0.0)
            idx = jnp.min(jnp.where(score <= rowmin, iota_row,
                                    jnp.float32(_M)),
                          axis=1, keepdims=True)
            w_mask = (iota_row == idx).astype(jnp.bfloat16)
            acc4 = acc4 + jax.lax.dot_general(
                sc4[:, lo:hi], w_mask,
                (((1,), (0,)), ((), ())),
                preferred_element_type=jnp.float32)

        colcnt = acc4[3:4, :]
        s1sum = [jnp.sum(colcnt * ptgtT[j:j + 1, :]) for j in range(3)]
        mu_t = [v * inv_n for v in s1sum]
        tcb = [_bf(ptgtT[j:j + 1, :] - mu_t[j]) for j in range(3)]
        H = [[jnp.sum(acc4[i:i + 1, :] * tcb[j]) for j in range(3)]
             for i in range(3)]
        R, t = _solve_rt(H, mu_s, mu_t)

        xb = _bf(temppc_ref[0:1, :])
        yb = _bf(temppc_ref[1:2, :])
        zb = _bf(temppc_ref[2:3, :])
        Rb = [[_bf(R[i][j]) for j in range(3)] for i in range(3)]
        temppc_ref[...] = jnp.concatenate(
            [((Rb[0][0] * xb + Rb[0][1] * yb) + Rb[0][2] * zb) + t[0],
             ((Rb[1][0] * xb + Rb[1][1] * yb) + Rb[1][2] * zb) + t[1],
             ((Rb[2][0] * xb + Rb[2][1] * yb) + Rb[2][2] * zb) + t[2]],
            axis=0)
        return carry_dummy

    jax.lax.fori_loop(0, _STEPS, step, jnp.int32(0))

    sum_p = [jnp.sum(psrcT[i:i + 1, :]) for i in range(3)]
    sum_c = [jnp.sum(temppc_ref[i:i + 1, :]) for i in range(3)]
    mu_p = [v * inv_n for v in sum_p]
    mu_c = [v * inv_n for v in sum_c]
    pcb = [_bf(psrcT[i:i + 1, :] - mu_p[i]) for i in range(3)]
    ccb = [_bf(temppc_ref[j:j + 1, :] - mu_c[j]) for j in range(3)]
    Hf = [[jnp.sum(pcb[i] * ccb[j]) for j in range(3)] for i in range(3)]
    R, t = _solve_rt(Hf, mu_p, mu_c)

    row_i = jax.lax.broadcasted_iota(jnp.int32, (3, 4), 0)
    col_i = jax.lax.broadcasted_iota(jnp.int32, (3, 4), 1)
    outmat = jnp.zeros((3, 4), jnp.float32)
    vals = [[R[0][0], R[0][1], R[0][2], t[0]],
            [R[1][0], R[1][1], R[1][2], t[1]],
            [R[2][0], R[2][1], R[2][2], t[2]]]
    for i in range(3):
        for j in range(4):
            outmat = outmat + vals[i][j] * jnp.where(
                (row_i == i) & (col_i == j), jnp.float32(1.0), jnp.float32(0.0))
    out_ref[0] = outmat


@functools.partial(jax.jit, static_argnames=("interpret",))
def _icp_pallas(psrc, ptgt, interpret=False):
    B = psrc.shape[0]
    psrcT = jnp.swapaxes(psrc, -1, -2)
    ptgtT = jnp.swapaxes(ptgt, -1, -2)
    return pl.pallas_call(
        _icp_body,
        grid=(B,),
        in_specs=[
            pl.BlockSpec((1, 3, _N), lambda b: (b, 0, 0)),
            pl.BlockSpec((1, 3, _M), lambda b: (b, 0, 0)),
        ],
        out_specs=pl.BlockSpec((1, 3, 4), lambda b: (b, 0, 0)),
        out_shape=jax.ShapeDtypeStruct((B, 3, 4), jnp.float32),
        scratch_shapes=[pltpu.VMEM((3, _N), jnp.float32)],
        compiler_params=pltpu.CompilerParams(
            dimension_semantics=("parallel",)),
        interpret=interpret,
    )(psrcT, ptgtT)


def kernel(psrc, ptgt):
    return _icp_pallas(psrc, ptgt)
